# Initial kernel scaffold; baseline (speedup 1.0000x reference)
import jax
import jax.numpy as jnp
from jax.experimental import pallas as pl


def kernel(x, samples, W_in, b_in, W_out, b_out):
    raise NotImplementedError("write your pallas kernel here")


# speedup vs baseline: 1.8803x; 1.8803x over previous
import functools

import jax
import jax.numpy as jnp
from jax import lax
from jax.experimental import pallas as pl
from jax.experimental.pallas import tpu as pltpu
from jax.experimental.pallas import tpu_sc as plsc

N = 320000
D = 128
S = 10000
S_PAD = 10240
NW = 16
CH = N // NW
SLICE = S_PAD // NW
L = 16

BLK = 6400


def _mv_body(x_ref, w_in_ref, b_in_ref, w_out_ref, b_out_ref, o_ref):
    w = jnp.dot(w_out_ref[...], w_in_ref[...],
                preferred_element_type=jnp.float32)
    c = jnp.sum(w_out_ref[...] * b_in_ref[...]) + b_out_ref[0, 0]
    o_ref[...] = jnp.sum(x_ref[...] * w, axis=1, keepdims=True) + c


def _matvec(x, w_in, b_in, w_out, b_out):
    return pl.pallas_call(
        _mv_body,
        grid=(N // BLK,),
        in_specs=[
            pl.BlockSpec((BLK, D), lambda i: (i, 0)),
            pl.BlockSpec((D, D), lambda i: (0, 0)),
            pl.BlockSpec((1, D), lambda i: (0, 0)),
            pl.BlockSpec((1, D), lambda i: (0, 0)),
            pl.BlockSpec((1, 1), lambda i: (0, 0)),
        ],
        out_specs=pl.BlockSpec((BLK, 1), lambda i: (i, 0)),
        out_shape=jax.ShapeDtypeStruct((N, 1), jnp.float32),
    )(x, w_in, b_in, w_out, b_out)


_GATHER_DNUMS = lax.GatherDimensionNumbers(
    offset_dims=(), collapsed_slice_dims=(0,), start_index_map=(0,))


def _g16(src, idx):
    return lax.gather(src, idx[:, None], _GATHER_DNUMS, slice_sizes=(1,),
                      mode=lax.GatherScatterMode.PROMISE_IN_BOUNDS)


def _sc_body(pred_hbm, samples_hbm, out_hbm,
             s_buf, v_buf, acc, pres, tmp_f, tmp_i, asl, psl, out_v,
             sums_sh, pres_sh, rsums_sh, rpres_sh):
    wid = lax.axis_index("s")
    base = wid * CH
    iota = lax.iota(jnp.int32, L)
    zeros_f = jnp.zeros((L,), jnp.float32)
    zeros_i = jnp.zeros((L,), jnp.int32)
    ones_i = jnp.ones((L,), jnp.int32)
    full15 = jnp.full((L,), 15, jnp.int32)

    def zero_body(j, _):
        acc[pl.ds(j * L, L)] = zeros_f
        pres[pl.ds(j * L, L)] = zeros_i
        return 0
    lax.fori_loop(0, S_PAD // L, zero_body, 0)

    pltpu.sync_copy(pred_hbm.at[pl.ds(base, CH)], v_buf)
    pltpu.sync_copy(samples_hbm.at[pl.ds(base, CH)], s_buf.at[pl.ds(0, CH)])

    @pl.when(wid < NW - 1)
    def _():
        pltpu.sync_copy(samples_hbm.at[pl.ds(base + CH, L)],
                        s_buf.at[pl.ds(CH, L)])

    @pl.when(wid == NW - 1)
    def _():
        s_buf[pl.ds(CH, L)] = jnp.full((L,), -1, jnp.int32)

    def scan_body(i, carry):
        s_cur = s_buf[pl.ds(i * L, L)]
        s_fut = s_buf[pl.ds(i * L + L, L)]
        v = v_buf[pl.ds(i * L, L)]
        s_next = jnp.where(iota < 15,
                           _g16(s_cur, jnp.minimum(iota + 1, 15)),
                           _g16(s_fut, zeros_i))
        ends = s_cur != s_next
        c = plsc.cumsum(v)
        idx_end = jnp.where(ends, iota, -1)
        m = plsc.cummax(idx_end)
        pe = jnp.where(iota >= 1, _g16(m, jnp.maximum(iota - 1, 0)), -1)
        c_pe = _g16(c, jnp.maximum(pe, 0))
        total = jnp.where(pe >= 0, c - c_pe, carry + c)
        plsc.addupdate_scatter(acc, [s_cur], total, mask=ends)
        plsc.addupdate_scatter(pres, [s_cur], ones_i, mask=ends)
        le = _g16(m, full15)
        c15 = _g16(c, full15)
        c_le = _g16(c, jnp.maximum(le, 0))
        return jnp.where(le >= 0, c15 - c_le, carry + c15)

    carry = lax.fori_loop(0, CH // L, scan_body, zeros_f)

    last_vec = s_buf[pl.ds(CH - L, L)]
    lv = _g16(last_vec, full15)
    plsc.addupdate_scatter(acc, [lv], carry, mask=iota == 0)

    pltpu.sync_copy(acc, sums_sh.at[wid])
    pltpu.sync_copy(pres, pres_sh.at[wid])
    plsc.subcore_barrier()

    off = wid * SLICE

    def bzero(j, _):
        asl[pl.ds(j * L, L)] = zeros_f
        psl[pl.ds(j * L, L)] = zeros_i
        return 0
    lax.fori_loop(0, SLICE // L, bzero, 0)

    def red_tile(t, _):
        pltpu.sync_copy(sums_sh.at[t, pl.ds(off, SLICE)], tmp_f)
        pltpu.sync_copy(pres_sh.at[t, pl.ds(off, SLICE)], tmp_i)

        def add_vec(j, _):
            asl[pl.ds(j * L, L)] = asl[pl.ds(j * L, L)] + tmp_f[pl.ds(j * L, L)]
            psl[pl.ds(j * L, L)] = psl[pl.ds(j * L, L)] + tmp_i[pl.ds(j * L, L)]
            return 0
        lax.fori_loop(0, SLICE // L, add_vec, 0)
        return 0
    lax.fori_loop(0, NW, red_tile, 0)

    pltpu.sync_copy(asl, rsums_sh.at[pl.ds(off, SLICE)])
    pltpu.sync_copy(psl, rpres_sh.at[pl.ds(off, SLICE)])
    plsc.subcore_barrier()

    @pl.when(wid == 0)
    def _():
        pltpu.sync_copy(rsums_sh, acc)
        pltpu.sync_copy(rpres_sh, pres)

        def czero(j, _):
            out_v[pl.ds(j * L, L)] = zeros_f
            return 0
        lax.fori_loop(0, S_PAD // L, czero, 0)

        def rank_body(j, run):
            p = pres[pl.ds(j * L, L)] > 0
            p01 = p.astype(jnp.int32)
            incl = plsc.cumsum(p01)
            rank = run + incl - p01
            sv = acc[pl.ds(j * L, L)]
            plsc.store_scatter(out_v, [rank], sv, mask=p)
            return run + _g16(incl, full15)

        lax.fori_loop(0, S_PAD // L, rank_body, zeros_i)
        pltpu.sync_copy(out_v, out_hbm)


@functools.partial(jax.jit, static_argnames=())
def _sc_segsum(pred, samples):
    mesh = plsc.VectorSubcoreMesh(core_axis_name="c", subcore_axis_name="s",
                                  num_cores=1, num_subcores=NW)
    f = pl.kernel(
        _sc_body,
        out_type=jax.ShapeDtypeStruct((S_PAD,), jnp.float32),
        mesh=mesh,
        compiler_params=pltpu.CompilerParams(needs_layout_passes=False),
        scratch_types=[
            pltpu.VMEM((CH + L,), jnp.int32),
            pltpu.VMEM((CH,), jnp.float32),
            pltpu.VMEM((S_PAD,), jnp.float32),
            pltpu.VMEM((S_PAD,), jnp.int32),
            pltpu.VMEM((SLICE,), jnp.float32),
            pltpu.VMEM((SLICE,), jnp.int32),
            pltpu.VMEM((SLICE,), jnp.float32),
            pltpu.VMEM((SLICE,), jnp.int32),
            pltpu.VMEM((S_PAD,), jnp.float32),
            pltpu.VMEM_SHARED((NW, S_PAD), jnp.float32),
            pltpu.VMEM_SHARED((NW, S_PAD), jnp.int32),
            pltpu.VMEM_SHARED((S_PAD,), jnp.float32),
            pltpu.VMEM_SHARED((S_PAD,), jnp.int32),
        ],
    )
    return f(pred, samples)


def kernel(x, samples, W_in, b_in, W_out, b_out):
    pred = _matvec(x, W_in, b_in.reshape(1, D), W_out, b_out.reshape(1, 1))
    out_pad = _sc_segsum(pred.reshape(N), samples)
    return out_pad[:S].reshape(S, 1)


# speedup vs baseline: 2.9220x; 1.5540x over previous
import functools

import jax
import jax.numpy as jnp
from jax import lax
from jax.experimental import pallas as pl
from jax.experimental.pallas import tpu as pltpu
from jax.experimental.pallas import tpu_sc as plsc

N = 320000
D = 128
S = 10000
S_PAD = 10240
NW = 16
CH = N // NW
SLICE = S_PAD // NW
L = 16

BLK = 6400


def _mv_body(x_ref, w_in_ref, b_in_ref, w_out_ref, b_out_ref, o_ref):
    w = jnp.dot(w_out_ref[...], w_in_ref[...],
                preferred_element_type=jnp.float32)
    c = jnp.sum(w_out_ref[...] * b_in_ref[...]) + b_out_ref[0, 0]
    o_ref[...] = (jnp.sum(x_ref[...] * w, axis=1) + c).reshape(1, BLK // 128, 128)


def _matvec(x, w_in, b_in, w_out, b_out):
    return pl.pallas_call(
        _mv_body,
        grid=(N // BLK,),
        in_specs=[
            pl.BlockSpec((BLK, D), lambda i: (i, 0)),
            pl.BlockSpec((D, D), lambda i: (0, 0)),
            pl.BlockSpec((1, D), lambda i: (0, 0)),
            pl.BlockSpec((1, D), lambda i: (0, 0)),
            pl.BlockSpec((1, 1), lambda i: (0, 0)),
        ],
        out_specs=pl.BlockSpec((1, BLK // 128, 128), lambda i: (i, 0, 0)),
        out_shape=jax.ShapeDtypeStruct((N // BLK, BLK // 128, 128), jnp.float32),
    )(x, w_in, b_in, w_out, b_out)


_GATHER_DNUMS = lax.GatherDimensionNumbers(
    offset_dims=(), collapsed_slice_dims=(0,), start_index_map=(0,))


def _g16(src, idx):
    return lax.gather(src, idx[:, None], _GATHER_DNUMS, slice_sizes=(1,),
                      mode=lax.GatherScatterMode.PROMISE_IN_BOUNDS)


def _sc_body(pred_hbm, samples_hbm, out_hbm,
             s_buf, v_buf, acc, pres, tmp_f, tmp_i, asl, psl, out_v,
             sums_sh, pres_sh, rsums_sh, rpres_sh):
    wid = lax.axis_index("s")
    base = wid * CH
    iota = lax.iota(jnp.int32, L)
    zeros_f = jnp.zeros((L,), jnp.float32)
    zeros_i = jnp.zeros((L,), jnp.int32)
    ones_i = jnp.ones((L,), jnp.int32)
    full15 = jnp.full((L,), 15, jnp.int32)

    def zero_body(j, _):
        acc[pl.ds(j * L, L)] = zeros_f
        pres[pl.ds(j * L, L)] = zeros_i
        return 0
    lax.fori_loop(0, S_PAD // L, zero_body, 0)

    pltpu.sync_copy(pred_hbm.at[pl.ds(base, CH)], v_buf)
    pltpu.sync_copy(samples_hbm.at[pl.ds(base, CH)], s_buf.at[pl.ds(0, CH)])

    @pl.when(wid < NW - 1)
    def _():
        pltpu.sync_copy(samples_hbm.at[pl.ds(base + CH, L)],
                        s_buf.at[pl.ds(CH, L)])

    @pl.when(wid == NW - 1)
    def _():
        s_buf[pl.ds(CH, L)] = jnp.full((L,), -1, jnp.int32)

    def scan_body(i, carry):
        s_cur = s_buf[pl.ds(i * L, L)]
        s_fut = s_buf[pl.ds(i * L + L, L)]
        v = v_buf[pl.ds(i * L, L)]
        s_next = jnp.where(iota < 15,
                           _g16(s_cur, jnp.minimum(iota + 1, 15)),
                           _g16(s_fut, zeros_i))
        ends = s_cur != s_next
        c = plsc.cumsum(v)
        idx_end = jnp.where(ends, iota, -1)
        m = plsc.cummax(idx_end)
        pe = jnp.where(iota >= 1, _g16(m, jnp.maximum(iota - 1, 0)), -1)
        c_pe = _g16(c, jnp.maximum(pe, 0))
        total = jnp.where(pe >= 0, c - c_pe, carry + c)
        plsc.addupdate_scatter(acc, [s_cur], total, mask=ends)
        plsc.addupdate_scatter(pres, [s_cur], ones_i, mask=ends)
        le = _g16(m, full15)
        c15 = _g16(c, full15)
        c_le = _g16(c, jnp.maximum(le, 0))
        return jnp.where(le >= 0, c15 - c_le, carry + c15)

    carry = lax.fori_loop(0, CH // L, scan_body, zeros_f)

    last_vec = s_buf[pl.ds(CH - L, L)]
    lv = _g16(last_vec, full15)
    plsc.addupdate_scatter(acc, [lv], carry, mask=iota == 0)

    pltpu.sync_copy(acc, sums_sh.at[wid])
    pltpu.sync_copy(pres, pres_sh.at[wid])
    plsc.subcore_barrier()

    off = wid * SLICE

    def bzero(j, _):
        asl[pl.ds(j * L, L)] = zeros_f
        psl[pl.ds(j * L, L)] = zeros_i
        return 0
    lax.fori_loop(0, SLICE // L, bzero, 0)

    def red_tile(t, _):
        pltpu.sync_copy(sums_sh.at[t, pl.ds(off, SLICE)], tmp_f)
        pltpu.sync_copy(pres_sh.at[t, pl.ds(off, SLICE)], tmp_i)

        def add_vec(j, _):
            asl[pl.ds(j * L, L)] = asl[pl.ds(j * L, L)] + tmp_f[pl.ds(j * L, L)]
            psl[pl.ds(j * L, L)] = psl[pl.ds(j * L, L)] + tmp_i[pl.ds(j * L, L)]
            return 0
        lax.fori_loop(0, SLICE // L, add_vec, 0)
        return 0
    lax.fori_loop(0, NW, red_tile, 0)

    pltpu.sync_copy(asl, rsums_sh.at[pl.ds(off, SLICE)])
    pltpu.sync_copy(psl, rpres_sh.at[pl.ds(off, SLICE)])
    plsc.subcore_barrier()

    @pl.when(wid == 0)
    def _():
        pltpu.sync_copy(rsums_sh, acc)
        pltpu.sync_copy(rpres_sh, pres)

        def czero(j, _):
            out_v[pl.ds(j * L, L)] = zeros_f
            return 0
        lax.fori_loop(0, S_PAD // L, czero, 0)

        def rank_body(j, run):
            p = pres[pl.ds(j * L, L)] > 0
            p01 = p.astype(jnp.int32)
            incl = plsc.cumsum(p01)
            rank = run + incl - p01
            sv = acc[pl.ds(j * L, L)]
            plsc.store_scatter(out_v, [rank], sv, mask=p)
            return run + _g16(incl, full15)

        lax.fori_loop(0, S_PAD // L, rank_body, zeros_i)
        pltpu.sync_copy(out_v, out_hbm)


@functools.partial(jax.jit, static_argnames=())
def _sc_segsum(pred, samples):
    mesh = plsc.VectorSubcoreMesh(core_axis_name="c", subcore_axis_name="s",
                                  num_cores=1, num_subcores=NW)
    f = pl.kernel(
        _sc_body,
        out_type=jax.ShapeDtypeStruct((S_PAD,), jnp.float32),
        mesh=mesh,
        compiler_params=pltpu.CompilerParams(needs_layout_passes=False),
        scratch_types=[
            pltpu.VMEM((CH + L,), jnp.int32),
            pltpu.VMEM((CH,), jnp.float32),
            pltpu.VMEM((S_PAD,), jnp.float32),
            pltpu.VMEM((S_PAD,), jnp.int32),
            pltpu.VMEM((SLICE,), jnp.float32),
            pltpu.VMEM((SLICE,), jnp.int32),
            pltpu.VMEM((SLICE,), jnp.float32),
            pltpu.VMEM((SLICE,), jnp.int32),
            pltpu.VMEM((S_PAD,), jnp.float32),
            pltpu.VMEM_SHARED((NW, S_PAD), jnp.float32),
            pltpu.VMEM_SHARED((NW, S_PAD), jnp.int32),
            pltpu.VMEM_SHARED((S_PAD,), jnp.float32),
            pltpu.VMEM_SHARED((S_PAD,), jnp.int32),
        ],
    )
    return f(pred, samples)


def kernel(x, samples, W_in, b_in, W_out, b_out):
    pred = _matvec(x, W_in, b_in.reshape(1, D), W_out, b_out.reshape(1, 1))
    out_pad = _sc_segsum(pred.reshape(N), samples)
    return out_pad[:S].reshape(S, 1)
